# baseline (device time: 11665 ns/iter reference)
import jax
import jax.numpy as jnp
from jax import lax
from jax.experimental import pallas as pl
from jax.experimental.pallas import tpu as pltpu

N_DEV = 16


def kernel(x):
    m, n = x.shape
    dtype = jnp.float32

    def body(x_ref, out_ref, send_buf, recv_bufs, send_sems, recv_sems):
        k = lax.axis_index("i")

        barrier_sem = pltpu.get_barrier_semaphore()
        for o in range(1, N_DEV):
            @pl.when(k >= o)
            def _(o=o):
                pl.semaphore_signal(
                    barrier_sem, inc=1,
                    device_id=(k - o,),
                    device_id_type=pl.DeviceIdType.MESH,
                )

        t = x_ref[:, :]
        size = m
        while size > 1:
            half = size // 2
            t = t[:half, :] * t[half:size, :]
            size = half
        send_buf[:, :] = t

        n_targets = (N_DEV - 1) - k
        pl.semaphore_wait(barrier_sem, n_targets)
        for o in range(1, N_DEV):
            @pl.when(k + o < N_DEV)
            def _(o=o):
                snd = pltpu.make_async_remote_copy(
                    src_ref=send_buf,
                    dst_ref=recv_bufs.at[o - 1],
                    send_sem=send_sems.at[o - 1],
                    recv_sem=recv_sems.at[o - 1],
                    device_id=(k + o,),
                    device_id_type=pl.DeviceIdType.MESH,
                )
                snd.start()

        y = x_ref[:, :]
        d = 1
        while d < m:
            y = y * jnp.concatenate(
                [jnp.ones((d, n), dtype), y[: m - d, :]], axis=0
            )
            d *= 2

        for o in range(1, N_DEV):
            @pl.when(k >= o)
            def _(o=o):
                rcv = pltpu.make_async_remote_copy(
                    src_ref=send_buf,
                    dst_ref=recv_bufs.at[o - 1],
                    send_sem=send_sems.at[o - 1],
                    recv_sem=recv_sems.at[o - 1],
                    device_id=(k,),
                    device_id_type=pl.DeviceIdType.MESH,
                )
                rcv.wait_recv()

        r = recv_bufs[:, 0, :]
        row = lax.broadcasted_iota(jnp.int32, (N_DEV - 1, n), 0)
        w = jnp.where(row < k, r, jnp.ones_like(r))
        w = jnp.concatenate([w, jnp.ones((1, n), dtype)], axis=0)
        size = N_DEV
        while size > 1:
            half = size // 2
            w = w[:half, :] * w[half:size, :]
            size = half

        out_ref[:, :] = y * w

        for o in range(1, N_DEV):
            @pl.when(k + o < N_DEV)
            def _(o=o):
                snd = pltpu.make_async_remote_copy(
                    src_ref=send_buf,
                    dst_ref=recv_bufs.at[o - 1],
                    send_sem=send_sems.at[o - 1],
                    recv_sem=recv_sems.at[o - 1],
                    device_id=(k + o,),
                    device_id_type=pl.DeviceIdType.MESH,
                )
                snd.wait_send()

    return pl.pallas_call(
        body,
        out_shape=jax.ShapeDtypeStruct((m, n), dtype),
        in_specs=[pl.BlockSpec(memory_space=pltpu.VMEM)],
        out_specs=pl.BlockSpec(memory_space=pltpu.VMEM),
        scratch_shapes=[
            pltpu.VMEM((1, n), dtype),
            pltpu.VMEM((N_DEV - 1, 1, n), dtype),
            pltpu.SemaphoreType.DMA((N_DEV - 1,)),
            pltpu.SemaphoreType.DMA((N_DEV - 1,)),
        ],
        compiler_params=pltpu.CompilerParams(collective_id=0),
    )(x)


# device time: 10475 ns/iter; 1.1136x vs baseline; 1.1136x over previous
import jax
import jax.numpy as jnp
from jax import lax
from jax.experimental import pallas as pl
from jax.experimental.pallas import tpu as pltpu

N_DEV = 16


def kernel(x):
    m, n = x.shape
    dtype = jnp.float32

    def body(x_ref, out_ref, send_buf, recv_bufs, send_sems, recv_sems):
        k = lax.axis_index("i")

        barrier_sem = pltpu.get_barrier_semaphore()
        for o in range(1, N_DEV):
            @pl.when(k >= o)
            def _(o=o):
                pl.semaphore_signal(
                    barrier_sem, inc=1,
                    device_id=(k - o,),
                    device_id_type=pl.DeviceIdType.MESH,
                )

        t = x_ref[:, :]
        size = m
        while size > 1:
            half = size // 2
            t = t[:half, :] * t[half:size, :]
            size = half
        send_buf[:, :] = t

        y = x_ref[:, :]
        d = 1
        while d <= 32:
            y = y * jnp.concatenate(
                [jnp.ones((d, n), dtype), y[: m - d, :]], axis=0
            )
            d *= 2

        n_targets = (N_DEV - 1) - k
        pl.semaphore_wait(barrier_sem, n_targets)
        for o in range(1, N_DEV):
            @pl.when(k + o < N_DEV)
            def _(o=o):
                snd = pltpu.make_async_remote_copy(
                    src_ref=send_buf,
                    dst_ref=recv_bufs.at[o - 1],
                    send_sem=send_sems.at[o - 1],
                    recv_sem=recv_sems.at[o - 1],
                    device_id=(k + o,),
                    device_id_type=pl.DeviceIdType.MESH,
                )
                snd.start()

        while d < m:
            y = y * jnp.concatenate(
                [jnp.ones((d, n), dtype), y[: m - d, :]], axis=0
            )
            d *= 2

        for o in range(1, N_DEV):
            @pl.when(k >= o)
            def _(o=o):
                rcv = pltpu.make_async_remote_copy(
                    src_ref=send_buf,
                    dst_ref=recv_bufs.at[o - 1],
                    send_sem=send_sems.at[o - 1],
                    recv_sem=recv_sems.at[o - 1],
                    device_id=(k,),
                    device_id_type=pl.DeviceIdType.MESH,
                )
                rcv.wait_recv()

        r = recv_bufs[:, 0, :]
        row = lax.broadcasted_iota(jnp.int32, (N_DEV - 1, n), 0)
        w = jnp.where(row < k, r, jnp.ones_like(r))
        w = jnp.concatenate([w, jnp.ones((1, n), dtype)], axis=0)
        size = N_DEV
        while size > 1:
            half = size // 2
            w = w[:half, :] * w[half:size, :]
            size = half

        out_ref[:, :] = y * w

        for o in range(1, N_DEV):
            @pl.when(k + o < N_DEV)
            def _(o=o):
                snd = pltpu.make_async_remote_copy(
                    src_ref=send_buf,
                    dst_ref=recv_bufs.at[o - 1],
                    send_sem=send_sems.at[o - 1],
                    recv_sem=recv_sems.at[o - 1],
                    device_id=(k + o,),
                    device_id_type=pl.DeviceIdType.MESH,
                )
                snd.wait_send()

    return pl.pallas_call(
        body,
        out_shape=jax.ShapeDtypeStruct((m, n), dtype),
        in_specs=[pl.BlockSpec(memory_space=pltpu.VMEM)],
        out_specs=pl.BlockSpec(memory_space=pltpu.VMEM),
        scratch_shapes=[
            pltpu.VMEM((1, n), dtype),
            pltpu.VMEM((N_DEV - 1, 1, n), dtype),
            pltpu.SemaphoreType.DMA((N_DEV - 1,)),
            pltpu.SemaphoreType.DMA((N_DEV - 1,)),
        ],
        compiler_params=pltpu.CompilerParams(collective_id=0),
    )(x)


# device time: 10386 ns/iter; 1.1231x vs baseline; 1.0086x over previous
import jax
import jax.numpy as jnp
from jax import lax
from jax.experimental import pallas as pl
from jax.experimental.pallas import tpu as pltpu

N_DEV = 16


def kernel(x):
    m, n = x.shape
    dtype = jnp.float32

    def body(x_ref, out_ref, send_buf, recv_bufs, send_sems, recv_sems):
        k = lax.axis_index("i")

        barrier_sem = pltpu.get_barrier_semaphore()
        for o in range(N_DEV - 1, 0, -1):
            @pl.when(k >= o)
            def _(o=o):
                pl.semaphore_signal(
                    barrier_sem, inc=1,
                    device_id=(k - o,),
                    device_id_type=pl.DeviceIdType.MESH,
                )

        t = x_ref[:, :]
        size = m
        while size > 1:
            half = size // 2
            t = t[:half, :] * t[half:size, :]
            size = half
        send_buf[:, :] = t

        y = x_ref[:, :]
        d = 1
        while d <= 32:
            y = y * jnp.concatenate(
                [jnp.ones((d, n), dtype), y[: m - d, :]], axis=0
            )
            d *= 2

        n_targets = (N_DEV - 1) - k
        pl.semaphore_wait(barrier_sem, n_targets)
        for o in range(N_DEV - 1, 0, -1):
            @pl.when(k + o < N_DEV)
            def _(o=o):
                snd = pltpu.make_async_remote_copy(
                    src_ref=send_buf,
                    dst_ref=recv_bufs.at[o - 1],
                    send_sem=send_sems.at[o - 1],
                    recv_sem=recv_sems.at[o - 1],
                    device_id=(k + o,),
                    device_id_type=pl.DeviceIdType.MESH,
                )
                snd.start()

        while d < m:
            y = y * jnp.concatenate(
                [jnp.ones((d, n), dtype), y[: m - d, :]], axis=0
            )
            d *= 2

        for o in range(1, N_DEV):
            @pl.when(k >= o)
            def _(o=o):
                rcv = pltpu.make_async_remote_copy(
                    src_ref=send_buf,
                    dst_ref=recv_bufs.at[o - 1],
                    send_sem=send_sems.at[o - 1],
                    recv_sem=recv_sems.at[o - 1],
                    device_id=(k,),
                    device_id_type=pl.DeviceIdType.MESH,
                )
                rcv.wait_recv()

        r = recv_bufs[:, 0, :]
        row = lax.broadcasted_iota(jnp.int32, (N_DEV - 1, n), 0)
        w = jnp.where(row < k, r, jnp.ones_like(r))
        w = jnp.concatenate([w, jnp.ones((1, n), dtype)], axis=0)
        size = N_DEV
        while size > 1:
            half = size // 2
            w = w[:half, :] * w[half:size, :]
            size = half

        out_ref[:, :] = y * w

        for o in range(1, N_DEV):
            @pl.when(k + o < N_DEV)
            def _(o=o):
                snd = pltpu.make_async_remote_copy(
                    src_ref=send_buf,
                    dst_ref=recv_bufs.at[o - 1],
                    send_sem=send_sems.at[o - 1],
                    recv_sem=recv_sems.at[o - 1],
                    device_id=(k + o,),
                    device_id_type=pl.DeviceIdType.MESH,
                )
                snd.wait_send()

    return pl.pallas_call(
        body,
        out_shape=jax.ShapeDtypeStruct((m, n), dtype),
        in_specs=[pl.BlockSpec(memory_space=pltpu.VMEM)],
        out_specs=pl.BlockSpec(memory_space=pltpu.VMEM),
        scratch_shapes=[
            pltpu.VMEM((1, n), dtype),
            pltpu.VMEM((N_DEV - 1, 1, n), dtype),
            pltpu.SemaphoreType.DMA((N_DEV - 1,)),
            pltpu.SemaphoreType.DMA((N_DEV - 1,)),
        ],
        compiler_params=pltpu.CompilerParams(collective_id=0),
    )(x)
